# baseline (device time: 419992 ns/iter reference)
import jax
import jax.numpy as jnp
from jax import lax
from jax.experimental import pallas as pl
from jax.experimental.pallas import tpu as pltpu

N_ROWS = 4096
N_COLS = 4096
QROWS = 1024
BLK = 256
JBLK = QROWS // BLK
N_SEM = 4 * JBLK


def kernel(partial, gamma):
    partial2d = partial.reshape(2 * N_ROWS, N_COLS)
    gamma2d = gamma.reshape(1, N_COLS)

    def body(partial_ref, gamma_ref, out_ref, remote_ref,
             mine_v, rem_v, out_v, mine_sems, rem_sems, out_sems,
             send_sems, recv_sems):
        x = lax.axis_index("x")
        y = lax.axis_index("y")
        z = lax.axis_index("z")
        my_base = y * N_ROWS
        nbr_base = (1 - y) * N_ROWS
        g = 2 * x + z
        q_own = QROWS * g
        q_diag = QROWS * (3 - g)
        q_x = QROWS * (2 * (1 - x) + z)
        q_z = QROWS * (2 * x + (1 - z))

        sends = []

        def y_send(qbase, j, slot):
            rows = qbase + j * BLK
            rdma = pltpu.make_async_remote_copy(
                src_ref=partial_ref.at[pl.ds(nbr_base + rows, BLK), :],
                dst_ref=remote_ref.at[pl.ds(rows, BLK), :],
                send_sem=send_sems.at[slot],
                recv_sem=recv_sems.at[slot],
                device_id=(x, 1 - y, z),
            )
            rdma.start()
            sends.append(rdma)

        def recv_wait(qbase, j, slot):
            rows = qbase + j * BLK
            recv = pltpu.make_async_remote_copy(
                src_ref=partial_ref.at[pl.ds(nbr_base + rows, BLK), :],
                dst_ref=remote_ref.at[pl.ds(rows, BLK), :],
                send_sem=send_sems.at[slot],
                recv_sem=recv_sems.at[slot],
                device_id=(x, 1 - y, z),
            )
            recv.wait_recv()

        def forward(j):
            rows = q_own + j * BLK
            for slot, dev in ((8 + j, (1 - x, y, z)), (12 + j, (x, y, 1 - z))):
                rdma = pltpu.make_async_remote_copy(
                    src_ref=remote_ref.at[pl.ds(rows, BLK), :],
                    dst_ref=remote_ref.at[pl.ds(rows, BLK), :],
                    send_sem=send_sems.at[slot],
                    recv_sem=recv_sems.at[slot],
                    device_id=dev,
                )
                rdma.start()
                sends.append(rdma)

        for j in range(JBLK):
            y_send(q_own, j, j)
        for j in range(JBLK):
            y_send(q_diag, j, 4 + j)

        entries = [(q_own, 0, 0), (q_own, 1, 1), (q_own, 2, 2), (q_own, 3, 3),
                   (q_x, 0, 8), (q_z, 0, 12),
                   (q_x, 1, 9), (q_z, 1, 13),
                   (q_x, 2, 10), (q_z, 2, 14),
                   (q_diag, 0, 4),
                   (q_x, 3, 11), (q_z, 3, 15),
                   (q_diag, 1, 5), (q_diag, 2, 6), (q_diag, 3, 7)]
        n_ent = len(entries)

        def mine_dma(k):
            qbase, j, _ = entries[k]
            rows = qbase + j * BLK
            cp = pltpu.make_async_copy(
                partial_ref.at[pl.ds(my_base + rows, BLK), :],
                mine_v.at[k % 2], mine_sems.at[k % 2])
            cp.start()
            return cp

        cp_mine = mine_dma(0)
        cp_outs = [None, None]
        for k, (qbase, j, slot) in enumerate(entries):
            rows = qbase + j * BLK
            recv_wait(qbase, j, slot)
            if slot < JBLK:
                forward(j)
            cp_rem = pltpu.make_async_copy(
                remote_ref.at[pl.ds(rows, BLK), :], rem_v, rem_sems.at[0])
            cp_rem.start()
            cp_next = mine_dma(k + 1) if k + 1 < n_ent else None
            cp_mine.wait()
            cp_rem.wait()

            s = mine_v[k % 2] + rem_v[...]
            ss = jnp.sum(s * s, axis=1, keepdims=True)
            rinv = lax.rsqrt(ss * (1.0 / N_COLS) + 1e-6)
            if cp_outs[k % 2] is not None:
                cp_outs[k % 2].wait()
            out_v[k % 2] = s * rinv * gamma_ref[...]
            cp_out = pltpu.make_async_copy(
                out_v.at[k % 2], out_ref.at[pl.ds(rows, BLK), :],
                out_sems.at[k % 2])
            cp_out.start()
            cp_outs[k % 2] = cp_out
            cp_mine = cp_next

        for cp in cp_outs:
            cp.wait()
        for rdma in sends:
            rdma.wait_send()

    out, _ = pl.pallas_call(
        body,
        out_shape=(
            jax.ShapeDtypeStruct((N_ROWS, N_COLS), jnp.float32),
            jax.ShapeDtypeStruct((N_ROWS, N_COLS), jnp.float32),
        ),
        in_specs=[
            pl.BlockSpec(memory_space=pl.ANY),
            pl.BlockSpec(memory_space=pltpu.VMEM),
        ],
        out_specs=(
            pl.BlockSpec(memory_space=pl.ANY),
            pl.BlockSpec(memory_space=pl.ANY),
        ),
        scratch_shapes=[
            pltpu.VMEM((2, BLK, N_COLS), jnp.float32),
            pltpu.VMEM((BLK, N_COLS), jnp.float32),
            pltpu.VMEM((2, BLK, N_COLS), jnp.float32),
            pltpu.SemaphoreType.DMA((2,)),
            pltpu.SemaphoreType.DMA((1,)),
            pltpu.SemaphoreType.DMA((2,)),
            pltpu.SemaphoreType.DMA((N_SEM,)),
            pltpu.SemaphoreType.DMA((N_SEM,)),
        ],
    )(partial2d, gamma2d)
    return out


# device time: 340005 ns/iter; 1.2353x vs baseline; 1.2353x over previous
import jax
import jax.numpy as jnp
from jax import lax
from jax.experimental import pallas as pl
from jax.experimental.pallas import tpu as pltpu

N_ROWS = 4096
N_COLS = 4096
QROWS = 1024
BLK = 256
JBLK = QROWS // BLK
N_SEM = 4 * JBLK


def kernel(partial, gamma):
    partial2d = partial.reshape(2 * N_ROWS, N_COLS)
    gamma2d = gamma.reshape(1, N_COLS)

    def body(partial_ref, gamma_ref, out_ref, remote_ref,
             mine_v, rem_v, out_v, mine_sems, rem_sems, out_sems,
             send_sems, recv_sems):
        x = lax.axis_index("x")
        y = lax.axis_index("y")
        z = lax.axis_index("z")
        my_base = y * N_ROWS
        nbr_base = (1 - y) * N_ROWS
        g = 2 * x + z
        q_own = QROWS * g
        q_diag = QROWS * (3 - g)
        q_x = QROWS * (2 * (1 - x) + z)
        q_z = QROWS * (2 * x + (1 - z))

        sends = []

        def y_send(qbase, j, slot):
            rows = qbase + j * BLK
            rdma = pltpu.make_async_remote_copy(
                src_ref=partial_ref.at[pl.ds(nbr_base + rows, BLK), :],
                dst_ref=remote_ref.at[pl.ds(rows, BLK), :],
                send_sem=send_sems.at[slot],
                recv_sem=recv_sems.at[slot],
                device_id=(x, 1 - y, z),
            )
            rdma.start()
            sends.append(rdma)

        def recv_wait(qbase, j, slot):
            rows = qbase + j * BLK
            recv = pltpu.make_async_remote_copy(
                src_ref=partial_ref.at[pl.ds(nbr_base + rows, BLK), :],
                dst_ref=remote_ref.at[pl.ds(rows, BLK), :],
                send_sem=send_sems.at[slot],
                recv_sem=recv_sems.at[slot],
                device_id=(x, 1 - y, z),
            )
            recv.wait_recv()

        def forward(rows, send_slot, recv_slot, dev):
            rdma = pltpu.make_async_remote_copy(
                src_ref=remote_ref.at[pl.ds(rows, BLK), :],
                dst_ref=remote_ref.at[pl.ds(rows, BLK), :],
                send_sem=send_sems.at[send_slot],
                recv_sem=recv_sems.at[recv_slot],
                device_id=dev,
            )
            rdma.start()
            sends.append(rdma)

        for j in range(JBLK):
            y_send(q_own, j, j)
        for k in range(2):
            y_send(q_diag, k, 4 + k)

        entries = [(q_own, 0, 0), (q_own, 1, 1), (q_own, 2, 2), (q_own, 3, 3),
                   (q_x, 0, 8), (q_z, 0, 12),
                   (q_x, 1, 9), (q_z, 1, 13),
                   (q_x, 2, 10), (q_z, 2, 14),
                   (q_x, 3, 11), (q_z, 3, 15),
                   (q_diag, 0, 4), (q_diag, 1, 5),
                   (q_diag, 2, 6), (q_diag, 3, 7)]
        n_ent = len(entries)

        def mine_dma(k):
            qbase, j, _ = entries[k]
            rows = qbase + j * BLK
            cp = pltpu.make_async_copy(
                partial_ref.at[pl.ds(my_base + rows, BLK), :],
                mine_v.at[k % 2], mine_sems.at[k % 2])
            cp.start()
            return cp

        cp_mine = mine_dma(0)
        cp_outs = [None, None]
        for k, (qbase, j, slot) in enumerate(entries):
            rows = qbase + j * BLK
            recv_wait(qbase, j, slot)
            if slot < JBLK:
                forward(rows, 6 + j, 8 + j, (1 - x, y, z))
                forward(rows, 10 + j, 12 + j, (x, y, 1 - z))
            elif slot == 14:
                forward(rows, 14, 6, (1 - x, y, z))
            elif slot == 11:
                forward(rows, 15, 7, (x, y, 1 - z))
            cp_rem = pltpu.make_async_copy(
                remote_ref.at[pl.ds(rows, BLK), :], rem_v, rem_sems.at[0])
            cp_rem.start()
            cp_next = mine_dma(k + 1) if k + 1 < n_ent else None
            cp_mine.wait()
            cp_rem.wait()

            s = mine_v[k % 2] + rem_v[...]
            ss = jnp.sum(s * s, axis=1, keepdims=True)
            rinv = lax.rsqrt(ss * (1.0 / N_COLS) + 1e-6)
            if cp_outs[k % 2] is not None:
                cp_outs[k % 2].wait()
            out_v[k % 2] = s * rinv * gamma_ref[...]
            cp_out = pltpu.make_async_copy(
                out_v.at[k % 2], out_ref.at[pl.ds(rows, BLK), :],
                out_sems.at[k % 2])
            cp_out.start()
            cp_outs[k % 2] = cp_out
            cp_mine = cp_next

        for cp in cp_outs:
            cp.wait()
        for rdma in sends:
            rdma.wait_send()

    out, _ = pl.pallas_call(
        body,
        out_shape=(
            jax.ShapeDtypeStruct((N_ROWS, N_COLS), jnp.float32),
            jax.ShapeDtypeStruct((N_ROWS, N_COLS), jnp.float32),
        ),
        in_specs=[
            pl.BlockSpec(memory_space=pl.ANY),
            pl.BlockSpec(memory_space=pltpu.VMEM),
        ],
        out_specs=(
            pl.BlockSpec(memory_space=pl.ANY),
            pl.BlockSpec(memory_space=pl.ANY),
        ),
        scratch_shapes=[
            pltpu.VMEM((2, BLK, N_COLS), jnp.float32),
            pltpu.VMEM((BLK, N_COLS), jnp.float32),
            pltpu.VMEM((2, BLK, N_COLS), jnp.float32),
            pltpu.SemaphoreType.DMA((2,)),
            pltpu.SemaphoreType.DMA((1,)),
            pltpu.SemaphoreType.DMA((2,)),
            pltpu.SemaphoreType.DMA((N_SEM,)),
            pltpu.SemaphoreType.DMA((N_SEM,)),
        ],
    )(partial2d, gamma2d)
    return out


# device time: 331143 ns/iter; 1.2683x vs baseline; 1.0268x over previous
import jax
import jax.numpy as jnp
from jax import lax
from jax.experimental import pallas as pl
from jax.experimental.pallas import tpu as pltpu

N_ROWS = 4096
N_COLS = 4096
QROWS = 1024
BLK = 256
SUB = 64
SPQ = QROWS // SUB
SPB = BLK // SUB

Y_DIAG = (0, 1, 2, 3, 4, 9)
X_RELAY = (5, 6, 7, 8, 10)
Z_RELAY = (12, 13, 14, 15, 11)
DIAG_SLOT = {d: 16 + p for p, d in enumerate(Y_DIAG)}
DIAG_SLOT.update({d: 22 + p for p, d in enumerate(X_RELAY)})
DIAG_SLOT.update({d: 27 + p for p, d in enumerate(Z_RELAY)})
N_SEM = 64


def kernel(partial, gamma):
    partial2d = partial.reshape(2 * N_ROWS, N_COLS)
    gamma2d = gamma.reshape(1, N_COLS)

    def body(partial_ref, gamma_ref, out_ref, remote_ref,
             mine_v, rem_v, out_v, mine_sems, rem_sems, out_sems,
             send_sems, recv_sems):
        x = lax.axis_index("x")
        y = lax.axis_index("y")
        z = lax.axis_index("z")
        my_base = y * N_ROWS
        nbr_base = (1 - y) * N_ROWS
        g = 2 * x + z
        q_own = QROWS * g
        q_diag = QROWS * (3 - g)
        q_x = QROWS * (2 * (1 - x) + z)
        q_z = QROWS * (2 * x + (1 - z))
        x_dev = (1 - x, y, z)
        z_dev = (x, y, 1 - z)

        sends = []

        def y_send(rows, slot):
            rdma = pltpu.make_async_remote_copy(
                src_ref=partial_ref.at[pl.ds(nbr_base + rows, SUB), :],
                dst_ref=remote_ref.at[pl.ds(rows, SUB), :],
                send_sem=send_sems.at[slot],
                recv_sem=recv_sems.at[slot],
                device_id=(x, 1 - y, z),
            )
            rdma.start()
            sends.append(rdma)

        def sub_wait(rows, slot):
            recv = pltpu.make_async_remote_copy(
                src_ref=partial_ref.at[pl.ds(nbr_base + rows, SUB), :],
                dst_ref=remote_ref.at[pl.ds(rows, SUB), :],
                send_sem=send_sems.at[slot],
                recv_sem=recv_sems.at[slot],
                device_id=(x, 1 - y, z),
            )
            recv.wait_recv()

        def fwd(rows, send_slot, recv_slot, dev):
            rdma = pltpu.make_async_remote_copy(
                src_ref=remote_ref.at[pl.ds(rows, SUB), :],
                dst_ref=remote_ref.at[pl.ds(rows, SUB), :],
                send_sem=send_sems.at[send_slot],
                recv_sem=recv_sems.at[recv_slot],
                device_id=dev,
            )
            rdma.start()
            sends.append(rdma)

        for s in range(SPQ):
            y_send(q_own + SUB * s, s)
        for p, d in enumerate(Y_DIAG):
            y_send(q_diag + SUB * d, 16 + p)

        def sub_actions(qbase, s):
            acts = []
            if qbase is q_own:
                acts.append((22 + s, 32 + s, x_dev))
                acts.append((38 + s, 48 + s, z_dev))
                slot = s
            elif qbase is q_x:
                slot = 32 + s
                if s in Z_RELAY:
                    p = Z_RELAY.index(s)
                    acts.append((59 + p, 27 + p, z_dev))
            elif qbase is q_z:
                slot = 48 + s
                if s in X_RELAY:
                    p = X_RELAY.index(s)
                    acts.append((54 + p, 22 + p, x_dev))
            else:
                slot = DIAG_SLOT[s]
            return slot, acts

        groups = [(q_own, 0), (q_own, 1), (q_own, 2), (q_own, 3),
                  (q_x, 0), (q_z, 0), (q_x, 1), (q_z, 1),
                  (q_x, 2), (q_z, 2), (q_x, 3), (q_z, 3),
                  (q_diag, 0), (q_diag, 1), (q_diag, 3), (q_diag, 2)]
        n_grp = len(groups)

        def mine_dma(k):
            qbase, j = groups[k]
            rows = qbase + j * BLK
            cp = pltpu.make_async_copy(
                partial_ref.at[pl.ds(my_base + rows, BLK), :],
                mine_v.at[k % 2], mine_sems.at[k % 2])
            cp.start()
            return cp

        cp_mine = mine_dma(0)
        cp_outs = [None, None]
        for k, (qbase, j) in enumerate(groups):
            rows = qbase + j * BLK
            for s in range(SPB * j, SPB * j + SPB):
                slot, acts = sub_actions(qbase, s)
                sub_wait(qbase + SUB * s, slot)
                for send_slot, recv_slot, dev in acts:
                    fwd(qbase + SUB * s, send_slot, recv_slot, dev)

            cp_rem = pltpu.make_async_copy(
                remote_ref.at[pl.ds(rows, BLK), :], rem_v, rem_sems.at[0])
            cp_rem.start()
            cp_next = mine_dma(k + 1) if k + 1 < n_grp else None
            cp_mine.wait()
            cp_rem.wait()

            s_ = mine_v[k % 2] + rem_v[...]
            ss = jnp.sum(s_ * s_, axis=1, keepdims=True)
            rinv = lax.rsqrt(ss * (1.0 / N_COLS) + 1e-6)
            if cp_outs[k % 2] is not None:
                cp_outs[k % 2].wait()
            out_v[k % 2] = s_ * rinv * gamma_ref[...]
            cp_out = pltpu.make_async_copy(
                out_v.at[k % 2], out_ref.at[pl.ds(rows, BLK), :],
                out_sems.at[k % 2])
            cp_out.start()
            cp_outs[k % 2] = cp_out
            cp_mine = cp_next

        for cp in cp_outs:
            cp.wait()
        for rdma in sends:
            rdma.wait_send()

    out, _ = pl.pallas_call(
        body,
        out_shape=(
            jax.ShapeDtypeStruct((N_ROWS, N_COLS), jnp.float32),
            jax.ShapeDtypeStruct((N_ROWS, N_COLS), jnp.float32),
        ),
        in_specs=[
            pl.BlockSpec(memory_space=pl.ANY),
            pl.BlockSpec(memory_space=pltpu.VMEM),
        ],
        out_specs=(
            pl.BlockSpec(memory_space=pl.ANY),
            pl.BlockSpec(memory_space=pl.ANY),
        ),
        scratch_shapes=[
            pltpu.VMEM((2, BLK, N_COLS), jnp.float32),
            pltpu.VMEM((BLK, N_COLS), jnp.float32),
            pltpu.VMEM((2, BLK, N_COLS), jnp.float32),
            pltpu.SemaphoreType.DMA((2,)),
            pltpu.SemaphoreType.DMA((1,)),
            pltpu.SemaphoreType.DMA((2,)),
            pltpu.SemaphoreType.DMA((N_SEM,)),
            pltpu.SemaphoreType.DMA((N_SEM,)),
        ],
    )(partial2d, gamma2d)
    return out
